# baseline (speedup 1.0000x reference)
import functools

import jax
import jax.numpy as jnp
from jax import lax
from jax.experimental import pallas as pl
from jax.experimental.pallas import tpu as pltpu

EMB = 8192
DIM = 256
BLK = 512


def _vq_body(x_ref, e_ref, fsq_ref, esq_ref, out_ref):
    x = x_ref[...]
    fsq = fsq_ref[...]
    sim = lax.dot_general(
        x, e_ref[...], (((1,), (0,)), ((), ())),
        preferred_element_type=jnp.float32,
    )
    d = (fsq + esq_ref[...]) - 2.0 * sim
    idx = jnp.argmin(d, axis=1).astype(jnp.int32)
    onehot = (
        lax.broadcasted_iota(jnp.int32, (x.shape[0], EMB), 1) == idx[:, None]
    ).astype(jnp.float32)
    q = lax.dot_general(
        onehot, e_ref[...], (((1,), (1,)), ((), ())),
        precision=lax.Precision.HIGHEST,
        preferred_element_type=jnp.float32,
    )
    out_ref[...] = q


@functools.partial(jax.jit, static_argnames=("interpret",))
def _vq(x_flat, embeddings, interpret=False):
    n = x_flat.shape[0]
    grid = n // BLK
    fsq = jnp.sum(x_flat**2, axis=1, keepdims=True)
    esq = jnp.sum(embeddings**2, axis=0, keepdims=True)
    return pl.pallas_call(
        _vq_body,
        grid=(grid,),
        in_specs=[
            pl.BlockSpec((BLK, DIM), lambda i: (i, 0)),
            pl.BlockSpec((DIM, EMB), lambda i: (0, 0)),
            pl.BlockSpec((BLK, 1), lambda i: (i, 0)),
            pl.BlockSpec((1, EMB), lambda i: (0, 0)),
        ],
        out_specs=pl.BlockSpec((BLK, DIM), lambda i: (i, 0)),
        out_shape=jax.ShapeDtypeStruct((n, DIM), jnp.float32),
        interpret=interpret,
    )(x_flat, embeddings, fsq, esq)


def kernel(x, embeddings):
    x_flat = jnp.reshape(x, (-1, DIM))
    q = _vq(x_flat, embeddings)
    return jnp.reshape(q, x.shape)


# speedup vs baseline: 2.2394x; 2.2394x over previous
import functools

import jax
import jax.numpy as jnp
from jax import lax
from jax.experimental import pallas as pl
from jax.experimental.pallas import tpu as pltpu

EMB = 8192
DIM = 256
BLK = 512


def _vq_body(x_ref, e_ref, fsq_ref, esq_ref, out_ref):
    x = x_ref[...]
    fsq = fsq_ref[...]
    sim = lax.dot_general(
        x, e_ref[...], (((1,), (0,)), ((), ())),
        preferred_element_type=jnp.float32,
    )
    d = (fsq + esq_ref[...]) - 2.0 * sim
    idx = jnp.argmin(d, axis=1).astype(jnp.int32)
    onehot = (
        lax.broadcasted_iota(jnp.int32, (x.shape[0], EMB), 1) == idx[:, None]
    ).astype(jnp.float32)
    q = lax.dot_general(
        onehot, e_ref[...], (((1,), (1,)), ((), ())),
        preferred_element_type=jnp.float32,
    )
    out_ref[...] = q


@functools.partial(jax.jit, static_argnames=("interpret",))
def _vq(x_flat, embeddings, interpret=False):
    n = x_flat.shape[0]
    grid = n // BLK
    fsq = jnp.sum(x_flat**2, axis=1, keepdims=True)
    esq = jnp.sum(embeddings**2, axis=0, keepdims=True)
    return pl.pallas_call(
        _vq_body,
        grid=(grid,),
        in_specs=[
            pl.BlockSpec((BLK, DIM), lambda i: (i, 0)),
            pl.BlockSpec((DIM, EMB), lambda i: (0, 0)),
            pl.BlockSpec((BLK, 1), lambda i: (i, 0)),
            pl.BlockSpec((1, EMB), lambda i: (0, 0)),
        ],
        out_specs=pl.BlockSpec((BLK, DIM), lambda i: (i, 0)),
        out_shape=jax.ShapeDtypeStruct((n, DIM), jnp.float32),
        interpret=interpret,
    )(x_flat, embeddings, fsq, esq)


def kernel(x, embeddings):
    x_flat = jnp.reshape(x, (-1, DIM))
    q = _vq(x_flat, embeddings)
    return jnp.reshape(q, x.shape)


# speedup vs baseline: 3.3693x; 1.5045x over previous
import functools

import jax
import jax.numpy as jnp
from jax import lax
from jax.experimental import pallas as pl
from jax.experimental.pallas import tpu as pltpu
from jax.experimental.pallas import tpu_sc as plsc

EMB = 8192
DIM = 256
NTOK = 32768
BLK = 512

NC = 2
NS = 16
NW = NC * NS
B_PER_W = NTOK // NW
CHB = 128
NCH = B_PER_W // CHB


def _argmin_body(x_ref, e_ref, fsq_ref, esq_ref, idx_ref):
    x = x_ref[...]
    sim = lax.dot_general(
        x, e_ref[...], (((1,), (0,)), ((), ())),
        preferred_element_type=jnp.float32,
    )
    d = (fsq_ref[...] + esq_ref[...]) - 2.0 * sim
    idx_ref[...] = jnp.argmin(d, axis=1).astype(jnp.int32)


@functools.partial(jax.jit, static_argnames=("interpret",))
def _argmin_idx(x_flat, embeddings, interpret=False):
    n = x_flat.shape[0]
    fsq = jnp.sum(x_flat**2, axis=1, keepdims=True)
    esq = jnp.sum(embeddings**2, axis=0, keepdims=True)
    return pl.pallas_call(
        _argmin_body,
        grid=(n // BLK,),
        in_specs=[
            pl.BlockSpec((BLK, DIM), lambda i: (i, 0)),
            pl.BlockSpec((DIM, EMB), lambda i: (0, 0)),
            pl.BlockSpec((BLK, 1), lambda i: (i, 0)),
            pl.BlockSpec((1, EMB), lambda i: (0, 0)),
        ],
        out_specs=pl.BlockSpec((BLK,), lambda i: (i,)),
        out_shape=jax.ShapeDtypeStruct((n,), jnp.int32),
        interpret=interpret,
    )(x_flat, embeddings, fsq, esq)


@functools.partial(
    pl.kernel,
    out_type=jax.ShapeDtypeStruct((NTOK, DIM), jnp.float32),
    mesh=plsc.VectorSubcoreMesh(
        core_axis_name="c", subcore_axis_name="s",
        num_cores=NC, num_subcores=NS,
    ),
    scratch_types=[
        pltpu.VMEM((NCH, CHB), jnp.int32),
        pltpu.VMEM((CHB, DIM), jnp.float32),
        pltpu.VMEM((CHB, DIM), jnp.float32),
        pltpu.SemaphoreType.DMA,
        pltpu.SemaphoreType.DMA,
    ],
)
def _sc_gather(table_hbm, idx_hbm, out_hbm, idx_v, rows0, rows1, sem0, sem1):
    wid = lax.axis_index("s") * NC + lax.axis_index("c")
    base = wid * B_PER_W
    pltpu.sync_copy(idx_hbm.at[wid], idx_v)
    rows = (rows0, rows1)
    sems = (sem0, sem1)
    cps = [pltpu.async_copy(table_hbm.at[idx_v.at[0]], rows0, sem0), None]
    for j in range(1, NCH + 1):
        if j < NCH:
            p = j & 1
            cps[p] = pltpu.async_copy(table_hbm.at[idx_v.at[j]], rows[p], sems[p])
        pp = (j - 1) & 1
        cps[pp].wait()
        pltpu.sync_copy(rows[pp], out_hbm.at[pl.ds(base + (j - 1) * CHB, CHB)])


def kernel(x, embeddings):
    x_flat = jnp.reshape(x, (-1, DIM))
    idx = _argmin_idx(x_flat, embeddings)
    table = embeddings.T
    q = _sc_gather(table, jnp.reshape(idx, (NW, NCH, CHB)))
    return jnp.reshape(q, x.shape)
